# baseline (speedup 1.0000x reference)
import functools
import math

import jax
import jax.numpy as jnp
from jax import lax
from jax.experimental import pallas as pl
from jax.experimental.pallas import tpu as pltpu
from jax.experimental.pallas import tpu_sc as plsc

_TAU = 1.0
_K = 8
_NS = 16
_L = 16


def _router_body(x_ref, w_ref, b_ref, g_ref, probs_ref, idx_ref, *,
                 n_experts):
    z = jnp.dot(x_ref[...], w_ref[...], preferred_element_type=jnp.float32)
    z = (z + b_ref[...] + g_ref[...]) * (1.0 / _TAU)

    iota_f = lax.broadcasted_iota(jnp.int32, z.shape, 1).astype(jnp.float32)
    vals = z
    top_v = []
    top_i = []
    for _ in range(_K):
        m = jnp.max(vals, axis=1, keepdims=True)
        idx = jnp.min(jnp.where(vals == m, iota_f, float(n_experts)), axis=1,
                      keepdims=True)
        top_v.append(m)
        top_i.append(idx)
        vals = jnp.where(iota_f == idx, -jnp.inf, vals)

    vtop = jnp.concatenate(top_v, axis=1)
    itop = jnp.concatenate(top_i, axis=1).astype(jnp.int32)
    e = jnp.exp(vtop - top_v[0])
    probs_ref[...] = e / jnp.sum(e, axis=1, keepdims=True)
    idx_ref[...] = itop


def _make_sc_count_loss(n_idx, n_experts, total):
    per_w = n_idx // _NS
    n_chunks = per_w // _L
    mean_count = total / n_experts
    ln2 = math.log(2.0)
    mesh = plsc.VectorSubcoreMesh(core_axis_name="c", subcore_axis_name="s",
                                  num_cores=1)

    hist_len = _L * n_experts

    @functools.partial(
        pl.kernel,
        mesh=mesh,
        out_type=jax.ShapeDtypeStruct((_L,), jnp.float32),
        scratch_types=[
            pltpu.VMEM((per_w,), jnp.int32),
            pltpu.VMEM((hist_len,), jnp.float32),
            pltpu.VMEM((hist_len,), jnp.float32),
            pltpu.VMEM((_L,), jnp.float32),
            pltpu.VMEM_SHARED((_NS, hist_len), jnp.float32),
        ],
        compiler_params=pltpu.CompilerParams(needs_layout_passes=False),
    )
    def sc_count_loss(idx_hbm, loss_hbm, idx_v, hist_v, tmp_v, loss_v,
                      shared):
        sid = lax.axis_index("s")
        lane = lax.iota(jnp.int32, _L)
        zeros16 = jnp.zeros((_L,), jnp.float32)
        ones16 = jnp.ones((_L,), jnp.float32)

        for j in range(hist_len // _L):
            hist_v[pl.ds(j * _L, _L)] = zeros16

        pltpu.sync_copy(idx_hbm.at[pl.ds(sid * per_w, per_w)], idx_v)

        def scatter_body(i, carry):
            v = idx_v[pl.ds(i * _L, _L)]
            plsc.addupdate_scatter(hist_v, [lane * n_experts + v], ones16)
            return carry

        lax.fori_loop(0, n_chunks, scatter_body, 0, unroll=8)

        pltpu.sync_copy(hist_v, shared.at[sid])
        plsc.subcore_barrier()

        @pl.when(sid == 0)
        def _loss():
            def merge_body(w, carry):
                pltpu.sync_copy(shared.at[w], tmp_v)
                for j in range(hist_len // _L):
                    sl = pl.ds(j * _L, _L)
                    hist_v[sl] = hist_v[sl] + tmp_v[sl]
                return carry

            lax.fori_loop(1, _NS, merge_body, 0)
            acc = zeros16
            for k in range(n_experts // _L):
                s = zeros16
                for r in range(_L):
                    s = s + hist_v[pl.ds(r * n_experts + k * _L, _L)]
                u = s * (1.0 / mean_count)
                bits = plsc.bitcast(u, jnp.int32)
                eu = ((bits >> 23) & 0xFF) - 127
                m = plsc.bitcast((bits & 0x7FFFFF) | 0x3F800000, jnp.float32)
                r_ = (m - 1.0) / (m + 1.0)
                r2 = r_ * r_
                lnm = 2.0 * r_ * (1.0 + r2 * (
                    1.0 / 3 + r2 * (1.0 / 5 + r2 * (
                        1.0 / 7 + r2 * (1.0 / 9 + r2 * (1.0 / 11))))))
                acc = acc + (eu.astype(jnp.float32) * ln2 + lnm)
            loss = -jnp.sum(acc) * (1.0 / (n_experts * n_experts))
            loss_v[...] = zeros16 + loss
            pltpu.sync_copy(loss_v, loss_hbm)

    return sc_count_loss


def kernel(x, W, b, gumbel):
    B, S, D = x.shape
    E = W.shape[1]
    T = B * S
    x2 = x.reshape(T, D)
    g2 = gumbel.reshape(T, E)
    b2 = b.reshape(1, E)

    bt = 1024
    while T % bt:
        bt //= 2
    grid = T // bt

    body = functools.partial(_router_body, n_experts=E)
    probs, idx = pl.pallas_call(
        body,
        grid=(grid,),
        in_specs=[
            pl.BlockSpec((bt, D), lambda i: (i, 0)),
            pl.BlockSpec((D, E), lambda i: (0, 0)),
            pl.BlockSpec((1, E), lambda i: (0, 0)),
            pl.BlockSpec((bt, E), lambda i: (i, 0)),
        ],
        out_specs=[
            pl.BlockSpec((bt, _K), lambda i: (i, 0)),
            pl.BlockSpec((bt, _K), lambda i: (i, 0)),
        ],
        out_shape=[
            jax.ShapeDtypeStruct((T, _K), jnp.float32),
            jax.ShapeDtypeStruct((T, _K), jnp.int32),
        ],
        compiler_params=pltpu.CompilerParams(
            dimension_semantics=("arbitrary",),
        ),
    )(x2, W, b2, g2)

    sc_fn = _make_sc_count_loss(T * _K, E, T * _K)
    loss16 = sc_fn(idx.reshape(T * _K))

    return (probs.reshape(B, S, _K), idx.reshape(B, S, _K), loss16[0])
